# Initial kernel scaffold; baseline (speedup 1.0000x reference)
import jax
import jax.numpy as jnp
from jax.experimental import pallas as pl


def kernel(feature_x, adj_norm, W0, a0, W21, a21, Wout, aout):
    raise NotImplementedError("write your pallas kernel here")


# speedup vs baseline: 1.8152x; 1.8152x over previous
import functools

import jax
import jax.numpy as jnp
from jax.experimental import pallas as pl

_ALPHA = 0.2
_F32 = jnp.float32


def _elu(x):
    return jnp.where(x > 0, x, jnp.exp(jnp.minimum(x, 0.0)) - 1.0)


def _proj1_body(nh, x_ref, w_ref, a12_ref, hall_ref, up_ref, vq_ref):
    h = jnp.dot(x_ref[...], w_ref[...], preferred_element_type=_F32)
    hall_ref[...] = h
    f12 = jnp.dot(h, a12_ref[...], preferred_element_type=_F32)
    f1 = f12[:, :nh]
    f2 = f12[:, nh:]
    up_ref[...] = jnp.concatenate([jnp.exp(-f1), jnp.exp(-_ALPHA * f1)], axis=1)
    vq_ref[...] = jnp.concatenate([jnp.exp(-f2), jnp.exp(-_ALPHA * f2)], axis=1)


def _attn1_body(nh, H, n2, adj_ref, hall_ref, up_ref, vqT_ref, w2_ref, b12_ref,
                ls_ref, h2_ref, up2_ref, vq2_ref):
    mask = adj_ref[...] != 0.0
    parts = []
    for i in range(nh):
        uv = up_ref[:, i:i + 1] * vqT_ref[i:i + 1, :]
        pq = up_ref[:, nh + i:nh + i + 1] * vqT_ref[nh + i:nh + i + 1, :]
        e = jnp.where(mask, jnp.where(uv < 1.0, uv, pq), 0.0)
        rs = jnp.sum(e, axis=1, keepdims=True)
        hp = jnp.dot(e, hall_ref[:, i * H:(i + 1) * H],
                     preferred_element_type=_F32)
        parts.append(_elu(hp / (rs + 1e-16)))
    lsb = jnp.concatenate(parts, axis=1)
    ls_ref[...] = lsb
    h2 = jnp.dot(lsb, w2_ref[...], preferred_element_type=_F32)
    h2_ref[...] = h2
    f12 = jnp.dot(h2, b12_ref[...], preferred_element_type=_F32)
    f1 = f12[:, :n2]
    f2 = f12[:, n2:]
    up2_ref[...] = jnp.concatenate([jnp.exp(-f1), jnp.exp(-_ALPHA * f1)], axis=1)
    vq2_ref[...] = jnp.concatenate([jnp.exp(-f2), jnp.exp(-_ALPHA * f2)], axis=1)


def _attn2_body(O1, O2, adj_ref, h2_ref, up2_ref, vq2T_ref, o1_ref, o2_ref):
    mask = adj_ref[...] != 0.0
    offs = (0, O1, O1 + O2)
    wids = (O1, O2, O2)
    hps = []
    for k in range(3):
        uv = up2_ref[:, k:k + 1] * vq2T_ref[k:k + 1, :]
        pq = up2_ref[:, 3 + k:3 + k + 1] * vq2T_ref[3 + k:3 + k + 1, :]
        e = jnp.where(mask, jnp.where(uv < 1.0, uv, pq), 0.0)
        rs = jnp.sum(e, axis=1, keepdims=True)
        hp = jnp.dot(e, h2_ref[:, offs[k]:offs[k] + wids[k]],
                     preferred_element_type=_F32)
        hps.append(hp / (rs + 1e-16))
    o1_ref[...] = _elu(hps[0])
    o2_ref[...] = _elu((hps[1] + hps[2]) * 0.5)


def kernel(feature_x, adj_norm, W0, a0, W21, a21, Wout, aout):
    N, D = feature_x.shape
    nh, _, H = W0.shape
    O1 = W21.shape[1]
    nout, _, O2 = Wout.shape
    Otot = O1 + nout * O2
    n2 = 1 + nout

    W0cat = jnp.transpose(W0, (1, 0, 2)).reshape(D, nh * H)
    eye = jnp.eye(nh, dtype=_F32)
    A1 = (a0[:, :H][:, :, None] * eye[:, None, :]).reshape(nh * H, nh)
    A2 = (a0[:, H:][:, :, None] * eye[:, None, :]).reshape(nh * H, nh)
    A12 = jnp.concatenate([A1, A2], axis=1)
    W2cat = jnp.concatenate([W21] + [Wout[i] for i in range(nout)], axis=1)
    B1 = jnp.zeros((Otot, n2), _F32)
    B2 = jnp.zeros((Otot, n2), _F32)
    B1 = B1.at[:O1, 0].set(a21[:O1])
    B2 = B2.at[:O1, 0].set(a21[O1:])
    for i in range(nout):
        lo = O1 + i * O2
        B1 = B1.at[lo:lo + O2, 1 + i].set(aout[i, :O2])
        B2 = B2.at[lo:lo + O2, 1 + i].set(aout[i, O2:])
    B12 = jnp.concatenate([B1, B2], axis=1)

    PB = 512
    hall, up, vq = pl.pallas_call(
        functools.partial(_proj1_body, nh),
        grid=(N // PB,),
        in_specs=[
            pl.BlockSpec((PB, D), lambda i: (i, 0)),
            pl.BlockSpec((D, nh * H), lambda i: (0, 0)),
            pl.BlockSpec((nh * H, 2 * nh), lambda i: (0, 0)),
        ],
        out_specs=[
            pl.BlockSpec((PB, nh * H), lambda i: (i, 0)),
            pl.BlockSpec((PB, 2 * nh), lambda i: (i, 0)),
            pl.BlockSpec((PB, 2 * nh), lambda i: (i, 0)),
        ],
        out_shape=[
            jax.ShapeDtypeStruct((N, nh * H), _F32),
            jax.ShapeDtypeStruct((N, 2 * nh), _F32),
            jax.ShapeDtypeStruct((N, 2 * nh), _F32),
        ],
    )(feature_x, W0cat, A12)
    vqT = vq.T

    BR = 256
    ls, h2, up2, vq2 = pl.pallas_call(
        functools.partial(_attn1_body, nh, H, n2),
        grid=(N // BR,),
        in_specs=[
            pl.BlockSpec((BR, N), lambda i: (i, 0)),
            pl.BlockSpec((N, nh * H), lambda i: (0, 0)),
            pl.BlockSpec((BR, 2 * nh), lambda i: (i, 0)),
            pl.BlockSpec((2 * nh, N), lambda i: (0, 0)),
            pl.BlockSpec((nh * H, Otot), lambda i: (0, 0)),
            pl.BlockSpec((Otot, 2 * n2), lambda i: (0, 0)),
        ],
        out_specs=[
            pl.BlockSpec((BR, nh * H), lambda i: (i, 0)),
            pl.BlockSpec((BR, Otot), lambda i: (i, 0)),
            pl.BlockSpec((BR, 2 * n2), lambda i: (i, 0)),
            pl.BlockSpec((BR, 2 * n2), lambda i: (i, 0)),
        ],
        out_shape=[
            jax.ShapeDtypeStruct((N, nh * H), _F32),
            jax.ShapeDtypeStruct((N, Otot), _F32),
            jax.ShapeDtypeStruct((N, 2 * n2), _F32),
            jax.ShapeDtypeStruct((N, 2 * n2), _F32),
        ],
    )(adj_norm, hall, up, vqT, W2cat, B12)
    vq2T = vq2.T

    o1, o2 = pl.pallas_call(
        functools.partial(_attn2_body, O1, O2),
        grid=(N // BR,),
        in_specs=[
            pl.BlockSpec((BR, N), lambda i: (i, 0)),
            pl.BlockSpec((N, Otot), lambda i: (0, 0)),
            pl.BlockSpec((BR, 2 * n2), lambda i: (i, 0)),
            pl.BlockSpec((2 * n2, N), lambda i: (0, 0)),
        ],
        out_specs=[
            pl.BlockSpec((BR, O1), lambda i: (i, 0)),
            pl.BlockSpec((BR, O2), lambda i: (i, 0)),
        ],
        out_shape=[
            jax.ShapeDtypeStruct((N, O1), _F32),
            jax.ShapeDtypeStruct((N, O2), _F32),
        ],
    )(adj_norm, h2, up2, vq2T)

    return (o1, o2, ls)


# speedup vs baseline: 1.8497x; 1.0190x over previous
import functools

import jax
import jax.numpy as jnp
from jax.experimental import pallas as pl

_ALPHA = 0.2
_F32 = jnp.float32
_BF16 = jnp.bfloat16


def _elu(x):
    return jnp.where(x > 0, x, jnp.exp(jnp.minimum(x, 0.0)) - 1.0)


def _proj1_body(nh, x_ref, w_ref, a12_ref, hall_ref, up_ref, vq_ref):
    h = jnp.dot(x_ref[...], w_ref[...], preferred_element_type=_F32)
    hall_ref[...] = h.astype(_BF16)
    f12 = jnp.dot(h, a12_ref[...], preferred_element_type=_F32)
    f1 = f12[:, :nh]
    f2 = f12[:, nh:]
    up_ref[...] = jnp.concatenate([jnp.exp(-f1), jnp.exp(-_ALPHA * f1)], axis=1)
    vq_ref[...] = jnp.concatenate([jnp.exp(-f2), jnp.exp(-_ALPHA * f2)], axis=1)


def _attn1_body(nh, H, n2, adj_ref, hall_ref, up_ref, vqT_ref, w2_ref, b12_ref,
                ls_ref, h2_ref, up2_ref, vq2_ref, mask8_ref):
    mask = adj_ref[...] != 0.0
    mask8_ref[...] = mask.astype(jnp.int8)
    parts = []
    for i in range(nh):
        uv = up_ref[:, i:i + 1] * vqT_ref[i:i + 1, :]
        pq = up_ref[:, nh + i:nh + i + 1] * vqT_ref[nh + i:nh + i + 1, :]
        e = jnp.where(mask, jnp.minimum(uv, pq), 0.0)
        rs = jnp.sum(e, axis=1, keepdims=True)
        hp = jnp.dot(e.astype(_BF16), hall_ref[:, i * H:(i + 1) * H],
                     preferred_element_type=_F32)
        parts.append(_elu(hp / (rs + 1e-16)))
    lsb = jnp.concatenate(parts, axis=1)
    ls_ref[...] = lsb
    h2 = jnp.dot(lsb, w2_ref[...], preferred_element_type=_F32)
    h2_ref[...] = h2.astype(_BF16)
    f12 = jnp.dot(h2, b12_ref[...], preferred_element_type=_F32)
    f1 = f12[:, :n2]
    f2 = f12[:, n2:]
    up2_ref[...] = jnp.concatenate([jnp.exp(-f1), jnp.exp(-_ALPHA * f1)], axis=1)
    vq2_ref[...] = jnp.concatenate([jnp.exp(-f2), jnp.exp(-_ALPHA * f2)], axis=1)


def _attn2_body(O1, O2, mask8_ref, h2_ref, up2_ref, vq2T_ref, o1_ref, o2_ref):
    mask = mask8_ref[...] != 0
    offs = (0, O1, O1 + O2)
    wids = (O1, O2, O2)
    hps = []
    for k in range(3):
        uv = up2_ref[:, k:k + 1] * vq2T_ref[k:k + 1, :]
        pq = up2_ref[:, 3 + k:3 + k + 1] * vq2T_ref[3 + k:3 + k + 1, :]
        e = jnp.where(mask, jnp.minimum(uv, pq), 0.0)
        rs = jnp.sum(e, axis=1, keepdims=True)
        hp = jnp.dot(e.astype(_BF16), h2_ref[:, offs[k]:offs[k] + wids[k]],
                     preferred_element_type=_F32)
        hps.append(hp / (rs + 1e-16))
    o1_ref[...] = _elu(hps[0])
    o2_ref[...] = _elu((hps[1] + hps[2]) * 0.5)


def kernel(feature_x, adj_norm, W0, a0, W21, a21, Wout, aout):
    N, D = feature_x.shape
    nh, _, H = W0.shape
    O1 = W21.shape[1]
    nout, _, O2 = Wout.shape
    Otot = O1 + nout * O2
    n2 = 1 + nout

    W0cat = jnp.transpose(W0, (1, 0, 2)).reshape(D, nh * H)
    eye = jnp.eye(nh, dtype=_F32)
    A1 = (a0[:, :H][:, :, None] * eye[:, None, :]).reshape(nh * H, nh)
    A2 = (a0[:, H:][:, :, None] * eye[:, None, :]).reshape(nh * H, nh)
    A12 = jnp.concatenate([A1, A2], axis=1)
    W2cat = jnp.concatenate([W21] + [Wout[i] for i in range(nout)], axis=1)
    B1 = jnp.zeros((Otot, n2), _F32)
    B2 = jnp.zeros((Otot, n2), _F32)
    B1 = B1.at[:O1, 0].set(a21[:O1])
    B2 = B2.at[:O1, 0].set(a21[O1:])
    for i in range(nout):
        lo = O1 + i * O2
        B1 = B1.at[lo:lo + O2, 1 + i].set(aout[i, :O2])
        B2 = B2.at[lo:lo + O2, 1 + i].set(aout[i, O2:])
    B12 = jnp.concatenate([B1, B2], axis=1)

    PB = 512
    hall, up, vq = pl.pallas_call(
        functools.partial(_proj1_body, nh),
        grid=(N // PB,),
        in_specs=[
            pl.BlockSpec((PB, D), lambda i: (i, 0)),
            pl.BlockSpec((D, nh * H), lambda i: (0, 0)),
            pl.BlockSpec((nh * H, 2 * nh), lambda i: (0, 0)),
        ],
        out_specs=[
            pl.BlockSpec((PB, nh * H), lambda i: (i, 0)),
            pl.BlockSpec((PB, 2 * nh), lambda i: (i, 0)),
            pl.BlockSpec((PB, 2 * nh), lambda i: (i, 0)),
        ],
        out_shape=[
            jax.ShapeDtypeStruct((N, nh * H), _BF16),
            jax.ShapeDtypeStruct((N, 2 * nh), _F32),
            jax.ShapeDtypeStruct((N, 2 * nh), _F32),
        ],
    )(feature_x, W0cat, A12)
    vqT = vq.T

    BR = 256
    ls, h2, up2, vq2, mask8 = pl.pallas_call(
        functools.partial(_attn1_body, nh, H, n2),
        grid=(N // BR,),
        in_specs=[
            pl.BlockSpec((BR, N), lambda i: (i, 0)),
            pl.BlockSpec((N, nh * H), lambda i: (0, 0)),
            pl.BlockSpec((BR, 2 * nh), lambda i: (i, 0)),
            pl.BlockSpec((2 * nh, N), lambda i: (0, 0)),
            pl.BlockSpec((nh * H, Otot), lambda i: (0, 0)),
            pl.BlockSpec((Otot, 2 * n2), lambda i: (0, 0)),
        ],
        out_specs=[
            pl.BlockSpec((BR, nh * H), lambda i: (i, 0)),
            pl.BlockSpec((BR, Otot), lambda i: (i, 0)),
            pl.BlockSpec((BR, 2 * n2), lambda i: (i, 0)),
            pl.BlockSpec((BR, 2 * n2), lambda i: (i, 0)),
            pl.BlockSpec((BR, N), lambda i: (i, 0)),
        ],
        out_shape=[
            jax.ShapeDtypeStruct((N, nh * H), _F32),
            jax.ShapeDtypeStruct((N, Otot), _BF16),
            jax.ShapeDtypeStruct((N, 2 * n2), _F32),
            jax.ShapeDtypeStruct((N, 2 * n2), _F32),
            jax.ShapeDtypeStruct((N, N), jnp.int8),
        ],
    )(adj_norm, hall, up, vqT, W2cat, B12)
    vq2T = vq2.T

    o1, o2 = pl.pallas_call(
        functools.partial(_attn2_body, O1, O2),
        grid=(N // BR,),
        in_specs=[
            pl.BlockSpec((BR, N), lambda i: (i, 0)),
            pl.BlockSpec((N, Otot), lambda i: (0, 0)),
            pl.BlockSpec((BR, 2 * n2), lambda i: (i, 0)),
            pl.BlockSpec((2 * n2, N), lambda i: (0, 0)),
        ],
        out_specs=[
            pl.BlockSpec((BR, O1), lambda i: (i, 0)),
            pl.BlockSpec((BR, O2), lambda i: (i, 0)),
        ],
        out_shape=[
            jax.ShapeDtypeStruct((N, O1), _F32),
            jax.ShapeDtypeStruct((N, O2), _F32),
        ],
    )(mask8, h2, up2, vq2T)

    return (o1, o2, ls)
